# baseline (speedup 1.0000x reference)
import functools

import jax
import jax.numpy as jnp
from jax import lax
from jax.experimental import pallas as pl
from jax.experimental.pallas import tpu as pltpu
from jax.experimental.pallas import tpu_sc as plsc

_B = 16384
_D = 26
_EMB = 64
_ROW = _D * _EMB
_NC = 2
_NS = 16
_NW = _NC * _NS
_BPW = _B // _NW
_R = 64
_NCHUNK = _BPW // _R
_DP = 32


def _sc_body(x_hbm, w_hbm, v_hbm, vflat_hbm, idx_hbm, emb_hbm, wide_hbm,
             xbuf, vbuf, vfbuf, wbuf, idxbuf, obuf, widebuf, wsm, sbuf, qbuf, sem):
    cid = lax.axis_index("c")
    sid = lax.axis_index("s")
    wid = sid * _NC + cid
    base = wid * _BPW

    pltpu.sync_copy(x_hbm.at[pl.ds(base * _D, _BPW * _D)], xbuf)
    pltpu.sync_copy(w_hbm, wbuf)
    pltpu.sync_copy(idx_hbm, idxbuf)
    pltpu.sync_copy(vflat_hbm, vfbuf)
    pltpu.async_copy(v_hbm.at[idxbuf], vbuf, sem).wait()

    lanes = lax.iota(jnp.int32, 16)
    wv0 = wbuf[pl.ds(0, 16)]
    wv1 = wbuf[pl.ds(16, 16)]
    s_lo = jnp.zeros((16,), jnp.float32)
    s_hi = jnp.zeros((16,), jnp.float32)
    q_lo = jnp.zeros((16,), jnp.float32)
    q_hi = jnp.zeros((16,), jnp.float32)
    for e in range(_EMB):
        ve_lo = plsc.load_gather(vfbuf, [lanes * _EMB + e])
        ve_hi = plsc.load_gather(vfbuf, [(lanes + 16) * _EMB + e])
        s_lo = s_lo + ve_lo
        s_hi = s_hi + ve_hi
        q_lo = q_lo + ve_lo * ve_lo
        q_hi = q_hi + ve_hi * ve_hi
    for d in range(_D):
        sbuf[d] = s_lo[d] if d < 16 else s_hi[d - 16]
        qbuf[d] = q_lo[d] if d < 16 else q_hi[d - 16]
        wsm[d] = wv0[d] if d < 16 else wv1[d - 16]
    zero16 = jnp.zeros((16,), jnp.float32)

    @pl.loop(0, _NCHUNK)
    def _chunk(g):
        row0 = g * _R
        for grp in range(_R // 16):
            rows = row0 + grp * 16 + lanes

            @pl.loop(0, _D, init_carry=(zero16, zero16, zero16))
            def _field(d, carry):
                o1, sv, qv = carry
                xv = plsc.load_gather(xbuf, [rows * _D + d])
                v0 = vbuf[d, pl.ds(0, 16)]
                v1 = vbuf[d, pl.ds(16, 16)]
                v2 = vbuf[d, pl.ds(32, 16)]
                v3 = vbuf[d, pl.ds(48, 16)]
                dbase = d * _EMB
                for k in range(16):
                    xs = xv[k]
                    ob = (grp * 16 + k) * _ROW + dbase
                    obuf[pl.ds(ob, 16)] = xs * v0
                    obuf[pl.ds(ob + 16, 16)] = xs * v1
                    obuf[pl.ds(ob + 32, 16)] = xs * v2
                    obuf[pl.ds(ob + 48, 16)] = xs * v3
                o1 = o1 + xv * wsm[d]
                sv = sv + xv * sbuf[d]
                qv = qv + (xv * xv) * qbuf[d]
                return (o1, sv, qv)

            o1, sv, qv = _field
            o2 = 0.5 * (sv * sv - qv)
            plsc.store_scatter(widebuf, [rows * 2], o1)
            plsc.store_scatter(widebuf, [rows * 2 + 1], o2)

        pltpu.sync_copy(obuf, emb_hbm.at[pl.ds((base + row0) * _ROW, _R * _ROW)])

    pltpu.sync_copy(widebuf, wide_hbm.at[pl.ds(base * 2, _BPW * 2)])


@functools.partial(
    pl.kernel,
    out_type=(
        jax.ShapeDtypeStruct((_B * _ROW,), jnp.float32),
        jax.ShapeDtypeStruct((_B * 2,), jnp.float32),
    ),
    mesh=plsc.VectorSubcoreMesh(core_axis_name="c", subcore_axis_name="s"),
    compiler_params=pltpu.CompilerParams(needs_layout_passes=False),
    scratch_types=[
        pltpu.VMEM((_BPW * _D,), jnp.float32),
        pltpu.VMEM((_DP, 128), jnp.float32),
        pltpu.VMEM((_DP * _EMB,), jnp.float32),
        pltpu.VMEM((_DP,), jnp.float32),
        pltpu.VMEM((_DP,), jnp.int32),
        pltpu.VMEM((_R * _ROW,), jnp.float32),
        pltpu.VMEM((_BPW * 2,), jnp.float32),
        pltpu.SMEM((_DP,), jnp.float32),
        pltpu.SMEM((_DP,), jnp.float32),
        pltpu.SMEM((_DP,), jnp.float32),
        pltpu.SemaphoreType.DMA,
    ],
)
def _wide_part_sc(x_hbm, w_hbm, v_hbm, vflat_hbm, idx_hbm, emb_hbm, wide_hbm,
                  xbuf, vbuf, vfbuf, wbuf, idxbuf, obuf, widebuf, wsm, sbuf, qbuf, sem):
    _sc_body(x_hbm, w_hbm, v_hbm, vflat_hbm, idx_hbm, emb_hbm, wide_hbm,
             xbuf, vbuf, vfbuf, wbuf, idxbuf, obuf, widebuf, wsm, sbuf, qbuf, sem)


@jax.jit
def kernel(inputs, W, V, embedding_lookup_index):
    x_flat = jnp.reshape(inputs, (-1,))
    w_pad = jnp.pad(W, (0, _DP - _D))
    idx_pad = jnp.pad(embedding_lookup_index.astype(jnp.int32), (0, _DP - _D))
    v_pad = jnp.pad(jnp.reshape(V, (-1,)), (0, (_DP - _D) * _EMB))
    v_wide = jnp.pad(V, ((0, 0), (0, 128 - _EMB)))
    emb_flat, wide_flat = _wide_part_sc(x_flat, w_pad, v_wide, v_pad, idx_pad)
    wide_output = jnp.reshape(wide_flat, (_B, 2))
    embedded_fields = jnp.reshape(emb_flat, (_B, _D, _EMB))
    return (wide_output, embedded_fields)
